# baseline (speedup 1.0000x reference)
import functools

import jax
import jax.numpy as jnp
from jax import lax
from jax.experimental import pallas as pl
from jax.experimental.pallas import tpu as pltpu
from jax.experimental.pallas import tpu_sc as plsc

_MIX = 4
_GROUP = 32


def _target_body(out_ref):
    i = lax.broadcasted_iota(jnp.int32, out_ref.shape, 0)
    j = lax.broadcasted_iota(jnp.int32, out_ref.shape, 1)
    out_ref[...] = (i // _GROUP) * _GROUP + (i % _GROUP + j) % _GROUP


def _make_permute(B, T, C, dtype):
    S = T // _MIX
    R = T * B
    mesh = plsc.VectorSubcoreMesh(core_axis_name="c", subcore_axis_name="s")
    n_workers = mesh.num_cores * mesh.num_subcores
    rows_per_w = R // n_workers
    chunk = 56
    n_chunks = rows_per_w // chunk
    n_slots = 3

    @functools.partial(
        pl.kernel,
        out_type=jax.ShapeDtypeStruct((R, C), dtype),
        mesh=mesh,
        scratch_types=[
            pltpu.VMEM((rows_per_w,), jnp.int32),
            pltpu.VMEM((n_slots, chunk, C), dtype),
            [pltpu.SemaphoreType.DMA] * n_slots,
            [pltpu.SemaphoreType.DMA] * n_slots,
        ],
    )
    def permute(p_hbm, out_hbm, idx, bufs, sem_g, sem_s):
        wid = lax.axis_index("s") * mesh.num_cores + lax.axis_index("c")
        r0 = wid * rows_per_w

        lane = lax.iota(jnp.int32, 16)
        groups_per_b = B // _GROUP
        u0 = r0 // _GROUP

        def gen_idx(j, carry):
            unit = u0 + j // 2
            half = j % 2
            t = unit // groups_per_b
            g = unit % groups_per_b
            q = t // S
            base = t * B + g * _GROUP
            rot = (lane + (half * 16 + q)) & (_GROUP - 1)
            idx[pl.ds(j * 16, 16)] = base + rot
            return carry

        lax.fori_loop(0, rows_per_w // 16, gen_idx, 0, unroll=False)

        def gather(i):
            return pltpu.make_async_copy(
                p_hbm.at[idx.at[pl.ds(i * chunk, chunk)]],
                bufs.at[i % n_slots],
                sem_g[i % n_slots],
            )

        def scatter(i):
            return pltpu.make_async_copy(
                bufs.at[i % n_slots],
                out_hbm.at[pl.ds(r0 + i * chunk, chunk)],
                sem_s[i % n_slots],
            )

        for i in range(min(n_slots - 1, n_chunks)):
            gather(i).start()
        for i in range(n_chunks):
            gather(i).wait()
            scatter(i).start()
            if i + n_slots - 1 < n_chunks:
                if i >= 1:
                    scatter(i - 1).wait()
                gather(i + n_slots - 1).start()
        for i in range(max(0, n_chunks - n_slots), n_chunks):
            scatter(i).wait()

    return permute


def kernel(patches):
    B, T, C = patches.shape
    assert B % _GROUP == 0 and T % _MIX == 0

    target = pl.pallas_call(
        _target_body,
        out_shape=jax.ShapeDtypeStruct((B, _MIX), jnp.int32),
    )()

    pt = jnp.transpose(patches, (1, 0, 2)).reshape(T * B, C)
    out2d = _make_permute(B, T, C, patches.dtype)(pt)
    out = jnp.transpose(out2d.reshape(T, B, C), (1, 0, 2))
    return (out, target)
